# baseline (speedup 1.0000x reference)
import jax
import jax.numpy as jnp
from jax import lax
from jax.experimental import pallas as pl
from jax.experimental.pallas import tpu as pltpu
from jax.experimental.pallas import tpu_sc as plsc

B, H, LT, WD, D = 256, 50, 20, 128, 256
NW = 32
PAIRS = B * H
PPW = PAIRS // NW
GP = 80
NG = PPW // GP
NLANE = WD // 16
TSTEP = 10


def _sc_pool_body(idx_hbm, table_hbm, out_hbm, idx_v, g0, g1, g2, g3, g4,
                  s0, s1, s2, s3, s4, osem):
    gbufs = (g0, g1, g2, g3, g4)
    gsems = (s0, s1, s2, s3, s4)
    cid = lax.axis_index("c")
    sid = lax.axis_index("s")
    wid = sid * 2 + cid
    pltpu.sync_copy(idx_hbm.at[wid], idx_v)

    zero = jnp.zeros((16,), jnp.float32)
    for g in range(NG):
        for r in range(GP):
            for c in range(NLANE):
                gbufs[g][r, pl.ds(c * 16, 16)] = zero

        def fire(t, carry, g=g):
            pltpu.async_copy(table_hbm.at[idx_v.at[g * LT + t]], gbufs[g],
                             gsems[g], add=True)
            return carry
        lax.fori_loop(0, LT, fire, 0)

    for g in range(NG):
        def drain(t, carry, g=g):
            pltpu.make_async_copy(table_hbm.at[idx_v.at[0]], gbufs[g],
                                  gsems[g]).wait()
            return carry
        lax.fori_loop(0, LT, drain, 0)
        pltpu.async_copy(gbufs[g], out_hbm.at[pl.ds(wid * PPW + g * GP, GP)],
                         osem)
    for g in range(NG):
        pltpu.make_async_copy(gbufs[0],
                              out_hbm.at[pl.ds(wid * PPW, GP)], osem).wait()


def _sc_pool(idx4, word_emb):
    return pl.kernel(
        _sc_pool_body,
        out_type=jax.ShapeDtypeStruct((PAIRS, WD), jnp.float32),
        mesh=plsc.VectorSubcoreMesh(core_axis_name="c", subcore_axis_name="s"),
        scratch_types=(
            [pltpu.VMEM((NG * LT, GP), jnp.int32)]
            + [pltpu.VMEM((GP, WD), jnp.float32)] * NG
            + [pltpu.SemaphoreType.DMA] * (NG + 1)
        ),
    )(idx4, word_emb)


INV_LT = 1.0 / (LT + 1e-8)


def _gru_body(pooled_ref, ue_ref, mask_ref, wn_ref, bn_ref, wcat_ref, bih_ref,
              bhh_ref, out_ref, h_ref):
    t = pl.program_id(0)

    @pl.when(t == 0)
    def _init():
        h_ref[...] = ue_ref[...]

    h = h_ref[...]
    slen = jnp.sum(mask_ref[...], axis=1, keepdims=True)
    wih = wcat_ref[0]
    whh = wcat_ref[1]
    for s in range(TSTEP):
        x = jnp.tanh(
            jnp.dot(pooled_ref[s] * INV_LT, wn_ref[...],
                    preferred_element_type=jnp.float32) + bn_ref[...])
        gi = jnp.dot(x, wih,
                     preferred_element_type=jnp.float32) + bih_ref[...]
        gh = jnp.dot(h, whh,
                     preferred_element_type=jnp.float32) + bhh_ref[...]
        r = jax.nn.sigmoid(gi[:, :D] + gh[:, :D])
        z = jax.nn.sigmoid(gi[:, D:2 * D] + gh[:, D:2 * D])
        n = jnp.tanh(gi[:, 2 * D:] + r * gh[:, 2 * D:])
        hn = (1.0 - z) * n + z * h
        keep = slen >= (t * TSTEP + s + 1).astype(jnp.float32)
        h = jnp.where(keep, hn, h)
    h_ref[...] = h

    @pl.when(t == H // TSTEP - 1)
    def _emit():
        for i in range(out_ref.shape[1] // D):
            out_ref[:, i * D:(i + 1) * D] = h


def _gru_call(pooled3, user_embedding, user_history_mask, wn, bn2, wcat,
              bih2, bhh2, NN):
    return pl.pallas_call(
        _gru_body,
        grid=(H // TSTEP,),
        in_specs=[
            pl.BlockSpec((TSTEP, B, WD), lambda t: (t, 0, 0)),
            pl.BlockSpec((B, D), lambda t: (0, 0)),
            pl.BlockSpec((B, H), lambda t: (0, 0)),
            pl.BlockSpec((WD, D), lambda t: (0, 0)),
            pl.BlockSpec((1, D), lambda t: (0, 0)),
            pl.BlockSpec((2, D, 3 * D), lambda t: (0, 0, 0)),
            pl.BlockSpec((1, 3 * D), lambda t: (0, 0)),
            pl.BlockSpec((1, 3 * D), lambda t: (0, 0)),
        ],
        out_specs=pl.BlockSpec((B, NN * D), lambda t: (0, 0)),
        out_shape=jax.ShapeDtypeStruct((B, NN * D), jnp.float32),
        scratch_shapes=[pltpu.VMEM((B, D), jnp.float32)],
    )(pooled3, user_embedding, user_history_mask, wn, bn2, wcat, bih2,
      bhh2)


def kernel(user_title_text, user_title_mask, user_title_entity,
           user_content_text, user_content_mask, user_content_entity,
           user_category, user_subCategory, user_history_mask,
           user_history_graph, user_history_category_mask,
           user_history_category_indices, user_embedding,
           candidate_news_representation, word_emb, W_news, b_news, W_ih,
           W_hh, b_ih, b_hh):
    NN = candidate_news_representation.shape[1]
    idx4 = (user_title_text.astype(jnp.int32)
            .transpose(1, 0, 2)
            .reshape(NW, NG, GP, LT)
            .transpose(0, 1, 3, 2)
            .reshape(NW, NG * LT, GP))
    pooled = _sc_pool(idx4, word_emb)
    pooled3 = pooled.reshape(H, B, WD)
    wcat = jnp.stack([W_ih, W_hh]).transpose(0, 2, 1)
    rep = _gru_call(pooled3, user_embedding, user_history_mask, W_news,
                    b_news.reshape(1, D), wcat, b_ih.reshape(1, 3 * D),
                    b_hh.reshape(1, 3 * D), NN)
    return rep.reshape(B, NN, D)


# speedup vs baseline: 1.0051x; 1.0051x over previous
import jax
import jax.numpy as jnp
from jax import lax
from jax.experimental import pallas as pl
from jax.experimental.pallas import tpu as pltpu
from jax.experimental.pallas import tpu_sc as plsc

B, H, LT, WD, D = 256, 50, 20, 128, 256
NW = 32
PAIRS = B * H
PPW = PAIRS // NW
GP = 80
NG = PPW // GP
NLANE = WD // 16
TSTEP = 10


def _sc_pool_body(idx_hbm, table_hbm, out_hbm, idx_v, g0, g1, g2, g3, g4,
                  s0, s1, s2, s3, s4, osem):
    gbufs = (g0, g1, g2, g3, g4)
    gsems = (s0, s1, s2, s3, s4)
    cid = lax.axis_index("c")
    sid = lax.axis_index("s")
    wid = sid * 2 + cid
    pltpu.sync_copy(idx_hbm.at[wid], idx_v)

    zero = jnp.zeros((16,), jnp.float32)
    for g in range(NG):
        for r in range(GP):
            for c in range(NLANE):
                gbufs[g][r, pl.ds(c * 16, 16)] = zero

        def fire(t, carry, g=g):
            pltpu.async_copy(table_hbm.at[idx_v.at[g * LT + t]], gbufs[g],
                             gsems[g], add=True)
            return carry
        lax.fori_loop(0, LT, fire, 0)

    for g in range(NG):
        def drain(t, carry, g=g):
            pltpu.make_async_copy(table_hbm.at[idx_v.at[0]], gbufs[g],
                                  gsems[g]).wait()
            return carry
        lax.fori_loop(0, LT, drain, 0)
        pltpu.async_copy(gbufs[g], out_hbm.at[pl.ds(wid * PPW + g * GP, GP)],
                         osem)
    for g in range(NG):
        pltpu.make_async_copy(gbufs[0],
                              out_hbm.at[pl.ds(wid * PPW, GP)], osem).wait()


def _sc_pool(idx4, word_emb):
    return pl.kernel(
        _sc_pool_body,
        out_type=jax.ShapeDtypeStruct((PAIRS, WD), jnp.float32),
        mesh=plsc.VectorSubcoreMesh(core_axis_name="c", subcore_axis_name="s"),
        scratch_types=(
            [pltpu.VMEM((NG * LT, GP), jnp.int32)]
            + [pltpu.VMEM((GP, WD), jnp.float32)] * NG
            + [pltpu.SemaphoreType.DMA] * (NG + 1)
        ),
    )(idx4, word_emb)


INV_LT = 1.0 / (LT + 1e-8)


def _gru_body(pooled_ref, ue_ref, mask_ref, wn_ref, bn_ref, wcat_ref, bih_ref,
              bhh_ref, out_ref, h_ref):
    t = pl.program_id(0)

    @pl.when(t == 0)
    def _init():
        h_ref[...] = ue_ref[...]

    h = h_ref[...]
    slen = jnp.sum(mask_ref[...], axis=1, keepdims=True)
    wih = wcat_ref[0]
    whh = wcat_ref[1]
    for s in range(TSTEP):
        x = jnp.tanh(
            jnp.dot(pooled_ref[s] * INV_LT, wn_ref[...],
                    preferred_element_type=jnp.float32) + bn_ref[...])
        gi = jnp.dot(x, wih,
                     preferred_element_type=jnp.float32) + bih_ref[...]
        gh = jnp.dot(h, whh,
                     preferred_element_type=jnp.float32) + bhh_ref[...]
        r = jax.nn.sigmoid(gi[:, :D] + gh[:, :D])
        z = jax.nn.sigmoid(gi[:, D:2 * D] + gh[:, D:2 * D])
        n = jnp.tanh(gi[:, 2 * D:] + r * gh[:, 2 * D:])
        hn = (1.0 - z) * n + z * h
        keep = slen >= (t * TSTEP + s + 1).astype(jnp.float32)
        h = jnp.where(keep, hn, h)
    h_ref[...] = h

    @pl.when(t == H // TSTEP - 1)
    def _emit():
        out_ref[...] = h


def _gru_call(pooled3, user_embedding, user_history_mask, wn, bn2, wcat,
              bih2, bhh2, NN):
    return pl.pallas_call(
        _gru_body,
        grid=(H // TSTEP,),
        in_specs=[
            pl.BlockSpec((TSTEP, B, WD), lambda t: (t, 0, 0)),
            pl.BlockSpec((B, D), lambda t: (0, 0)),
            pl.BlockSpec((B, H), lambda t: (0, 0)),
            pl.BlockSpec((WD, D), lambda t: (0, 0)),
            pl.BlockSpec((1, D), lambda t: (0, 0)),
            pl.BlockSpec((2, D, 3 * D), lambda t: (0, 0, 0)),
            pl.BlockSpec((1, 3 * D), lambda t: (0, 0)),
            pl.BlockSpec((1, 3 * D), lambda t: (0, 0)),
        ],
        out_specs=pl.BlockSpec((B, D), lambda t: (0, 0)),
        out_shape=jax.ShapeDtypeStruct((B, D), jnp.float32),
        scratch_shapes=[pltpu.VMEM((B, D), jnp.float32)],
    )(pooled3, user_embedding, user_history_mask, wn, bn2, wcat, bih2,
      bhh2)


def kernel(user_title_text, user_title_mask, user_title_entity,
           user_content_text, user_content_mask, user_content_entity,
           user_category, user_subCategory, user_history_mask,
           user_history_graph, user_history_category_mask,
           user_history_category_indices, user_embedding,
           candidate_news_representation, word_emb, W_news, b_news, W_ih,
           W_hh, b_ih, b_hh):
    NN = candidate_news_representation.shape[1]
    idx4 = (user_title_text.astype(jnp.int32)
            .transpose(1, 0, 2)
            .reshape(NW, NG, GP, LT)
            .transpose(0, 1, 3, 2)
            .reshape(NW, NG * LT, GP))
    pooled = _sc_pool(idx4, word_emb)
    pooled3 = pooled.reshape(H, B, WD)
    wcat = jnp.stack([W_ih, W_hh]).transpose(0, 2, 1)
    rep = _gru_call(pooled3, user_embedding, user_history_mask, W_news,
                    b_news.reshape(1, D), wcat, b_ih.reshape(1, 3 * D),
                    b_hh.reshape(1, 3 * D), NN)
    return jnp.broadcast_to(rep[:, None, :], (B, NN, D))


# speedup vs baseline: 1.0058x; 1.0007x over previous
import jax
import jax.numpy as jnp
from jax import lax
from jax.experimental import pallas as pl
from jax.experimental.pallas import tpu as pltpu
from jax.experimental.pallas import tpu_sc as plsc

B, H, LT, WD, D = 256, 50, 20, 128, 256
NW = 32
PAIRS = B * H
PPW = PAIRS // NW
GP = 80
NG = PPW // GP
NLANE = WD // 16
TSTEP = 25


def _sc_pool_body(idx_hbm, table_hbm, out_hbm, idx_v, g0, g1, g2, g3, g4,
                  s0, s1, s2, s3, s4, osem):
    gbufs = (g0, g1, g2, g3, g4)
    gsems = (s0, s1, s2, s3, s4)
    cid = lax.axis_index("c")
    sid = lax.axis_index("s")
    wid = sid * 2 + cid
    pltpu.sync_copy(idx_hbm.at[wid], idx_v)

    zero = jnp.zeros((16,), jnp.float32)
    for g in range(NG):
        for r in range(GP):
            for c in range(NLANE):
                gbufs[g][r, pl.ds(c * 16, 16)] = zero

        def fire(t, carry, g=g):
            pltpu.async_copy(table_hbm.at[idx_v.at[g * LT + t]], gbufs[g],
                             gsems[g], add=True)
            return carry
        lax.fori_loop(0, LT, fire, 0)

    for g in range(NG):
        def drain(t, carry, g=g):
            pltpu.make_async_copy(table_hbm.at[idx_v.at[0]], gbufs[g],
                                  gsems[g]).wait()
            return carry
        lax.fori_loop(0, LT, drain, 0)
        pltpu.async_copy(gbufs[g], out_hbm.at[pl.ds(wid * PPW + g * GP, GP)],
                         osem)
    for g in range(NG):
        pltpu.make_async_copy(gbufs[0],
                              out_hbm.at[pl.ds(wid * PPW, GP)], osem).wait()


def _sc_pool(idx4, word_emb):
    return pl.kernel(
        _sc_pool_body,
        out_type=jax.ShapeDtypeStruct((PAIRS, WD), jnp.float32),
        mesh=plsc.VectorSubcoreMesh(core_axis_name="c", subcore_axis_name="s"),
        scratch_types=(
            [pltpu.VMEM((NG * LT, GP), jnp.int32)]
            + [pltpu.VMEM((GP, WD), jnp.float32)] * NG
            + [pltpu.SemaphoreType.DMA] * (NG + 1)
        ),
    )(idx4, word_emb)


INV_LT = 1.0 / (LT + 1e-8)


def _gru_body(pooled_ref, ue_ref, mask_ref, wn_ref, bn_ref, wcat_ref, bih_ref,
              bhh_ref, out_ref, h_ref):
    t = pl.program_id(0)

    @pl.when(t == 0)
    def _init():
        h_ref[...] = ue_ref[...]

    h = h_ref[...]
    slen = jnp.sum(mask_ref[...], axis=1, keepdims=True)
    wih = wcat_ref[0]
    whh = wcat_ref[1]
    for s in range(TSTEP):
        x = jnp.tanh(
            jnp.dot(pooled_ref[s] * INV_LT, wn_ref[...],
                    preferred_element_type=jnp.float32) + bn_ref[...])
        gi = jnp.dot(x, wih,
                     preferred_element_type=jnp.float32) + bih_ref[...]
        gh = jnp.dot(h, whh,
                     preferred_element_type=jnp.float32) + bhh_ref[...]
        r = jax.nn.sigmoid(gi[:, :D] + gh[:, :D])
        z = jax.nn.sigmoid(gi[:, D:2 * D] + gh[:, D:2 * D])
        n = jnp.tanh(gi[:, 2 * D:] + r * gh[:, 2 * D:])
        hn = (1.0 - z) * n + z * h
        keep = slen >= (t * TSTEP + s + 1).astype(jnp.float32)
        h = jnp.where(keep, hn, h)
    h_ref[...] = h

    @pl.when(t == H // TSTEP - 1)
    def _emit():
        out_ref[...] = h


def _gru_call(pooled3, user_embedding, user_history_mask, wn, bn2, wcat,
              bih2, bhh2, NN):
    return pl.pallas_call(
        _gru_body,
        grid=(H // TSTEP,),
        in_specs=[
            pl.BlockSpec((TSTEP, B, WD), lambda t: (t, 0, 0)),
            pl.BlockSpec((B, D), lambda t: (0, 0)),
            pl.BlockSpec((B, H), lambda t: (0, 0)),
            pl.BlockSpec((WD, D), lambda t: (0, 0)),
            pl.BlockSpec((1, D), lambda t: (0, 0)),
            pl.BlockSpec((2, D, 3 * D), lambda t: (0, 0, 0)),
            pl.BlockSpec((1, 3 * D), lambda t: (0, 0)),
            pl.BlockSpec((1, 3 * D), lambda t: (0, 0)),
        ],
        out_specs=pl.BlockSpec((B, D), lambda t: (0, 0)),
        out_shape=jax.ShapeDtypeStruct((B, D), jnp.float32),
        scratch_shapes=[pltpu.VMEM((B, D), jnp.float32)],
    )(pooled3, user_embedding, user_history_mask, wn, bn2, wcat, bih2,
      bhh2)


def kernel(user_title_text, user_title_mask, user_title_entity,
           user_content_text, user_content_mask, user_content_entity,
           user_category, user_subCategory, user_history_mask,
           user_history_graph, user_history_category_mask,
           user_history_category_indices, user_embedding,
           candidate_news_representation, word_emb, W_news, b_news, W_ih,
           W_hh, b_ih, b_hh):
    NN = candidate_news_representation.shape[1]
    idx4 = (user_title_text.astype(jnp.int32)
            .transpose(1, 0, 2)
            .reshape(NW, NG, GP, LT)
            .transpose(0, 1, 3, 2)
            .reshape(NW, NG * LT, GP))
    pooled = _sc_pool(idx4, word_emb)
    pooled3 = pooled.reshape(H, B, WD)
    wcat = jnp.stack([W_ih, W_hh]).transpose(0, 2, 1)
    rep = _gru_call(pooled3, user_embedding, user_history_mask, W_news,
                    b_news.reshape(1, D), wcat, b_ih.reshape(1, 3 * D),
                    b_hh.reshape(1, 3 * D), NN)
    return jnp.broadcast_to(rep[:, None, :], (B, NN, D))


# speedup vs baseline: 1.0113x; 1.0055x over previous
import jax
import jax.numpy as jnp
from jax import lax
from jax.experimental import pallas as pl
from jax.experimental.pallas import tpu as pltpu
from jax.experimental.pallas import tpu_sc as plsc

B, H, LT, WD, D = 256, 50, 20, 128, 256
NW = 32
PAIRS = B * H
PPW = PAIRS // NW
GP = 80
NG = PPW // GP
NLANE = WD // 16
TSTEP = 10


def _sc_pool_body(idx_hbm, table_hbm, out_hbm, idx_v, g0, g1, g2, g3, g4,
                  s0, s1, s2, s3, s4, osem):
    gbufs = (g0, g1, g2, g3, g4)
    gsems = (s0, s1, s2, s3, s4)
    cid = lax.axis_index("c")
    sid = lax.axis_index("s")
    wid = sid * 2 + cid
    pltpu.sync_copy(idx_hbm.at[wid], idx_v)

    zero = jnp.zeros((16,), jnp.float32)
    for g in range(NG):
        for r in range(GP):
            for c in range(NLANE):
                gbufs[g][r, pl.ds(c * 16, 16)] = zero

        def fire(t, carry, g=g):
            pltpu.async_copy(table_hbm.at[idx_v.at[g * LT + t]], gbufs[g],
                             gsems[g], add=True)
            return carry
        lax.fori_loop(0, LT, fire, 0)

    for g in range(NG):
        def drain(t, carry, g=g):
            pltpu.make_async_copy(table_hbm.at[idx_v.at[0]], gbufs[g],
                                  gsems[g]).wait()
            return carry
        lax.fori_loop(0, LT, drain, 0)
        pltpu.async_copy(gbufs[g], out_hbm.at[pl.ds(wid * PPW + g * GP, GP)],
                         osem)
    for g in range(NG):
        pltpu.make_async_copy(gbufs[0],
                              out_hbm.at[pl.ds(wid * PPW, GP)], osem).wait()


def _sc_pool(idx4, word_emb):
    return pl.kernel(
        _sc_pool_body,
        out_type=jax.ShapeDtypeStruct((PAIRS, WD), jnp.float32),
        mesh=plsc.VectorSubcoreMesh(core_axis_name="c", subcore_axis_name="s"),
        scratch_types=(
            [pltpu.VMEM((NG * LT, GP), jnp.int32)]
            + [pltpu.VMEM((GP, WD), jnp.float32)] * NG
            + [pltpu.SemaphoreType.DMA] * (NG + 1)
        ),
    )(idx4, word_emb)


INV_LT = 1.0 / (LT + 1e-8)


def _gru_body(pooled_ref, ue_ref, mask_ref, wn_ref, bn_ref, wcat_ref, bih_ref,
              bhh_ref, out_ref, h_ref):
    t = pl.program_id(0)

    @pl.when(t == 0)
    def _init():
        h_ref[...] = ue_ref[...]

    h = h_ref[...]
    slen = jnp.sum(mask_ref[...], axis=1, keepdims=True)
    wih = wcat_ref[0]
    whh = wcat_ref[1]
    for s in range(TSTEP):
        x = jnp.tanh(
            jnp.dot(pooled_ref[s] * INV_LT, wn_ref[...],
                    preferred_element_type=jnp.float32) + bn_ref[...])
        gi = jnp.dot(x, wih,
                     preferred_element_type=jnp.float32) + bih_ref[...]
        gh = jnp.dot(h, whh,
                     preferred_element_type=jnp.float32) + bhh_ref[...]
        r = jax.nn.sigmoid(gi[:, :D] + gh[:, :D])
        z = jax.nn.sigmoid(gi[:, D:2 * D] + gh[:, D:2 * D])
        n = jnp.tanh(gi[:, 2 * D:] + r * gh[:, 2 * D:])
        hn = (1.0 - z) * n + z * h
        keep = slen >= (t * TSTEP + s + 1).astype(jnp.float32)
        h = jnp.where(keep, hn, h)
    h_ref[...] = h

    @pl.when(t == H // TSTEP - 1)
    def _emit():
        out_ref[...] = h


def _gru_call(pooled3, user_embedding, user_history_mask, wn, bn2, wcat,
              bih2, bhh2, NN):
    return pl.pallas_call(
        _gru_body,
        grid=(H // TSTEP,),
        in_specs=[
            pl.BlockSpec((TSTEP, B, WD), lambda t: (t, 0, 0)),
            pl.BlockSpec((B, D), lambda t: (0, 0)),
            pl.BlockSpec((B, H), lambda t: (0, 0)),
            pl.BlockSpec((WD, D), lambda t: (0, 0)),
            pl.BlockSpec((1, D), lambda t: (0, 0)),
            pl.BlockSpec((2, D, 3 * D), lambda t: (0, 0, 0)),
            pl.BlockSpec((1, 3 * D), lambda t: (0, 0)),
            pl.BlockSpec((1, 3 * D), lambda t: (0, 0)),
        ],
        out_specs=pl.BlockSpec((B, D), lambda t: (0, 0)),
        out_shape=jax.ShapeDtypeStruct((B, D), jnp.float32),
        scratch_shapes=[pltpu.VMEM((B, D), jnp.float32)],
    )(pooled3, user_embedding, user_history_mask, wn, bn2, wcat, bih2,
      bhh2)


def kernel(user_title_text, user_title_mask, user_title_entity,
           user_content_text, user_content_mask, user_content_entity,
           user_category, user_subCategory, user_history_mask,
           user_history_graph, user_history_category_mask,
           user_history_category_indices, user_embedding,
           candidate_news_representation, word_emb, W_news, b_news, W_ih,
           W_hh, b_ih, b_hh):
    NN = candidate_news_representation.shape[1]
    idx4 = (user_title_text.astype(jnp.int32)
            .transpose(1, 0, 2)
            .reshape(NW, NG, GP, LT)
            .transpose(0, 1, 3, 2)
            .reshape(NW, NG * LT, GP))
    pooled = _sc_pool(idx4, word_emb)
    pooled3 = pooled.reshape(H, B, WD)
    wcat = jnp.stack([W_ih, W_hh]).transpose(0, 2, 1)
    rep = _gru_call(pooled3, user_embedding, user_history_mask, W_news,
                    b_news.reshape(1, D), wcat, b_ih.reshape(1, 3 * D),
                    b_hh.reshape(1, 3 * D), NN)
    return jnp.broadcast_to(rep[:, None, :], (B, NN, D))
